# Initial kernel scaffold; baseline (speedup 1.0000x reference)
import functools

import jax
import jax.numpy as jnp
import numpy as np
from jax import lax
from jax.experimental import pallas as pl
from jax.experimental.pallas import tpu as pltpu
from jax.experimental.pallas import tpu_sc as plsc

_N = 10000
_E = 160000
_HID = 256
_NH = 8
_DH = 32
_G = 16
_NC = 2
_NS = 16
_KCH = 128
_NCHUNK = _E // _KCH
_BE = 800
_NB = _E // _BE
_NZ = _N // _NS

_M8 = (np.arange(_HID)[:, None] // _DH == np.arange(_NH)[None, :]).astype(np.float32)
_R8 = np.ascontiguousarray(_M8.T)


def _sc_mesh():
    return plsc.VectorSubcoreMesh(core_axis_name="c", subcore_axis_name="s")


def _sc_gather(q, kv, dst, src):
    @functools.partial(
        pl.kernel,
        out_type=(
            jax.ShapeDtypeStruct((_E, _HID), jnp.float32),
            jax.ShapeDtypeStruct((_E, 2 * _HID), jnp.float32),
        ),
        mesh=_sc_mesh(),
        scratch_types=[
            pltpu.VMEM((_KCH,), jnp.int32),
            pltpu.VMEM((_KCH,), jnp.int32),
            pltpu.VMEM((_KCH, _HID), jnp.float32),
            pltpu.VMEM((_KCH, 2 * _HID), jnp.float32),
            pltpu.SemaphoreType.DMA,
            pltpu.SemaphoreType.DMA,
        ],
    )
    def k(q_hbm, kv_hbm, dst_hbm, src_hbm, qd_hbm, kvs_hbm, di, si, bq, bkv, s1, s2):
        wid = lax.axis_index("s") * _NC + lax.axis_index("c")
        nper = (_NCHUNK + _NC * _NS - 1) // (_NC * _NS)

        @pl.loop(0, nper)
        def _(j):
            g = j * (_NC * _NS) + wid

            @pl.when(g < _NCHUNK)
            def _():
                base = g * _KCH
                pltpu.sync_copy(dst_hbm.at[pl.ds(base, _KCH)], di)
                pltpu.sync_copy(src_hbm.at[pl.ds(base, _KCH)], si)
                cq = pltpu.async_copy(q_hbm.at[di], bq, s1)
                ckv = pltpu.async_copy(kv_hbm.at[si], bkv, s2)
                cq.wait()
                ckv.wait()
                pltpu.sync_copy(bq, qd_hbm.at[pl.ds(base, _KCH)])
                pltpu.sync_copy(bkv, kvs_hbm.at[pl.ds(base, _KCH)])

    return k(q, kv, dst, src)


def _sc_scatter(msg, wpad, dst):
    @functools.partial(
        pl.kernel,
        out_type=(
            jax.ShapeDtypeStruct((_NC, _N, 128), jnp.float32),
            jax.ShapeDtypeStruct((_N, 16), jnp.float32),
        ),
        mesh=_sc_mesh(),
        scratch_types=[
            pltpu.VMEM((_KCH,), jnp.int32),
            pltpu.VMEM((_KCH, 128), jnp.float32),
            pltpu.VMEM((_KCH, 16), jnp.float32),
            pltpu.VMEM((125, 128), jnp.float32),
            pltpu.VMEM((125, 16), jnp.float32),
            pltpu.VMEM_SHARED((_N, 128), jnp.float32),
            pltpu.VMEM_SHARED((_N, 16), jnp.float32),
        ],
    )
    def k(msg_hbm, w_hbm, dst_hbm, agg_hbm, den_hbm, di, bm, bw, bz, bzw, acc, dacc):
        cid = lax.axis_index("c")
        sid = lax.axis_index("s")

        zero16 = jnp.zeros((16,), jnp.float32)

        @pl.loop(0, 125)
        def _(r):
            @pl.loop(0, 8)
            def _(c):
                bz.at[r, pl.ds(c * 16, 16)][...] = zero16

            bzw.at[r, pl.ds(0, 16)][...] = zero16

        @pl.loop(0, 5)
        def _(t):
            row = sid * _NZ + t * 125
            pltpu.sync_copy(bz, acc.at[pl.ds(row, 125)])
            pltpu.sync_copy(bzw, dacc.at[pl.ds(row, 125)])

        plsc.subcore_barrier()

        nper = (_NCHUNK + _NS - 1) // _NS

        @pl.loop(0, nper)
        def _(j):
            g = j * _NS + sid

            @pl.when(g < _NCHUNK)
            def _():
                base = g * _KCH
                pltpu.sync_copy(dst_hbm.at[pl.ds(base, _KCH)], di)
                pltpu.sync_copy(msg_hbm.at[cid, pl.ds(base, _KCH)], bm)
                pltpu.sync_copy(bm, acc.at[di], add=True)

                @pl.when(cid == 0)
                def _():
                    pltpu.sync_copy(w_hbm.at[pl.ds(base, _KCH)], bw)
                    pltpu.sync_copy(bw, dacc.at[di], add=True)

        plsc.subcore_barrier()

        @pl.loop(0, 5)
        def _(t):
            row = sid * _NZ + t * 125
            pltpu.sync_copy(acc.at[pl.ds(row, 125)], agg_hbm.at[cid, pl.ds(row, 125)])

            @pl.when(cid == 0)
            def _():
                pltpu.sync_copy(dacc.at[pl.ds(row, 125)], den_hbm.at[pl.ds(row, 125)])

    return k(msg, wpad, dst)


def _edge_body(first, last, refs):
    (ein_ref, qd_ref, kvs_ref, we_ref, m8_ref, r8_ref, ew_ref) = refs[:7]
    rest = refs[7:]
    if last:
        msg_ref, w_ref = rest
        woe_ref = u_ref = st_ref = None
    else:
        woe_ref, msg_ref, w_ref, u_ref, st_ref = rest

    if first:
        e = jnp.dot(ein_ref[...], ew_ref[...], preferred_element_type=jnp.float32)
    else:
        e = ein_ref[...] * ew_ref[0:1, :] + ew_ref[1:2, :]
    ee = jnp.dot(e, we_ref[...], preferred_element_type=jnp.float32)
    kvs = kvs_ref[...]
    score = (qd_ref[...] * kvs[:, :_HID]) * ee * np.float32(1.0 / np.sqrt(_DH))
    s8 = jnp.dot(score, m8_ref[...], preferred_element_type=jnp.float32)
    w = jnp.exp(jnp.clip(s8, -5.0, 5.0))
    wx = jnp.dot(w, r8_ref[...], preferred_element_type=jnp.float32)
    m = wx * kvs[:, _HID:]
    msg_ref[0, :, :] = m[:, :128]
    msg_ref[1, :, :] = m[:, 128:]
    w_ref[...] = jnp.concatenate([w, jnp.zeros((_BE, 8), jnp.float32)], axis=1)
    if not last:
        u = e + jnp.dot(score, woe_ref[...], preferred_element_type=jnp.float32)
        u_ref[...] = u
        st_ref[...] = jnp.stack([jnp.sum(u, axis=0), jnp.sum(u * u, axis=0)])[None]


def _edge(first, last, ein, qd, kvs, ew, We, Woe):
    cin = ein.shape[1]
    in_specs = [
        pl.BlockSpec((_BE, cin), lambda b: (b, 0)),
        pl.BlockSpec((_BE, _HID), lambda b: (b, 0)),
        pl.BlockSpec((_BE, 2 * _HID), lambda b: (b, 0)),
        pl.BlockSpec((_HID, _HID), lambda b: (0, 0)),
        pl.BlockSpec((_HID, _NH), lambda b: (0, 0)),
        pl.BlockSpec((_NH, _HID), lambda b: (0, 0)),
        pl.BlockSpec(ew.shape, lambda b: (0, 0)),
    ]
    args = [ein, qd, kvs, We, jnp.asarray(_M8), jnp.asarray(_R8), ew]
    if not last:
        in_specs.append(pl.BlockSpec((_HID, _HID), lambda b: (0, 0)))
        args.append(Woe)
    out_shape = [
        jax.ShapeDtypeStruct((2, _E, 128), jnp.float32),
        jax.ShapeDtypeStruct((_E, 16), jnp.float32),
    ]
    out_specs = [
        pl.BlockSpec((2, _BE, 128), lambda b: (0, b, 0)),
        pl.BlockSpec((_BE, 16), lambda b: (b, 0)),
    ]
    if not last:
        out_shape += [
            jax.ShapeDtypeStruct((_E, _HID), jnp.float32),
            jax.ShapeDtypeStruct((_NB, 2, _HID), jnp.float32),
        ]
        out_specs += [
            pl.BlockSpec((_BE, _HID), lambda b: (b, 0)),
            pl.BlockSpec((1, 2, _HID), lambda b: (b, 0, 0)),
        ]

    return pl.pallas_call(
        lambda *refs: _edge_body(first, last, refs),
        grid=(_NB,),
        in_specs=in_specs,
        out_specs=out_specs,
        out_shape=out_shape,
        compiler_params=pltpu.CompilerParams(
            dimension_semantics=("parallel",),
        ),
    )(*args)


def _node_body(emit_escale, refs):
    if emit_escale:
        (h_ref, agg_ref, den_ref, st_ref, wo_ref, n1g, n1b, wf1, bf1, wf2, bf2,
         n2g, n2b, e1g, e1b, r8_ref, hout_ref, es_ref, h1_ref, t2_ref) = refs
    else:
        (h_ref, agg_ref, den_ref, wo_ref, n1g, n1b, wf1, bf1, wf2, bf2,
         n2g, n2b, r8_ref, hout_ref, h1_ref, t2_ref) = refs

    aggc = jnp.concatenate([agg_ref[0], agg_ref[1]], axis=1)
    denx = jnp.dot(den_ref[...][:, :_NH] + np.float32(1e-16), r8_ref[...],
                   preferred_element_type=jnp.float32)
    h1in = h_ref[...] + jnp.dot(aggc / denx, wo_ref[...],
                                preferred_element_type=jnp.float32)
    mu = jnp.mean(h1in, axis=0, keepdims=True)
    d = h1in - mu
    var = jnp.mean(d * d, axis=0, keepdims=True)
    h1 = d * (n1g[...] / jnp.sqrt(var + np.float32(1e-5))) + n1b[...]
    h1_ref[...] = h1

    nblk = 10
    bs = _N // nblk

    def ff_body(i, carry):
        hb = h1_ref[pl.ds(i * bs, bs), :]
        ff = jnp.maximum(
            jnp.dot(hb, wf1[...], preferred_element_type=jnp.float32) + bf1[...], 0.0)
        t2_ref[pl.ds(i * bs, bs), :] = (
            hb + jnp.dot(ff, wf2[...], preferred_element_type=jnp.float32) + bf2[...])
        return carry

    lax.fori_loop(0, nblk, ff_body, 0)

    t2 = t2_ref[...]
    m2 = jnp.mean(t2, axis=0, keepdims=True)
    d2 = t2 - m2
    v2 = jnp.mean(d2 * d2, axis=0, keepdims=True)
    hout_ref[...] = d2 * (n2g[...] / jnp.sqrt(v2 + np.float32(1e-5))) + n2b[...]

    if emit_escale:
        su = jnp.sum(st_ref[:, 0, :], axis=0)
        sq = jnp.sum(st_ref[:, 1, :], axis=0)
        me = su * np.float32(1.0 / _E)
        ve = sq * np.float32(1.0 / _E) - me * me
        s = e1g[0, :] / jnp.sqrt(ve + np.float32(1e-5))
        es_ref[...] = jnp.stack([s, e1b[0, :] - me * s])


def _node(h, agg, den, stats, Wo, n1g, n1b, wf1, bf1, wf2, bf2, n2g, n2b,
          e1g, e1b):
    emit_escale = stats is not None
    args = [h, agg, den]
    if emit_escale:
        args.append(stats)
    args += [Wo, n1g, n1b, wf1, bf1, wf2, bf2, n2g, n2b]
    if emit_escale:
        args += [e1g, e1b]
    args.append(jnp.asarray(_R8))
    out_shape = [jax.ShapeDtypeStruct((_N, _HID), jnp.float32)]
    if emit_escale:
        out_shape.append(jax.ShapeDtypeStruct((2, _HID), jnp.float32))
    res = pl.pallas_call(
        lambda *refs: _node_body(emit_escale, refs),
        out_shape=out_shape,
        scratch_shapes=[
            pltpu.VMEM((_N, _HID), jnp.float32),
            pltpu.VMEM((_N, _HID), jnp.float32),
        ],
    )(*args)
    if emit_escale:
        return res[0], res[1]
    return res[0], None


def _embed(x, pe, wn, wp):
    def body(x_ref, pe_ref, wn_ref, wp_ref, o_ref):
        o_ref[...] = (
            jnp.dot(x_ref[...], wn_ref[...], preferred_element_type=jnp.float32)
            + jnp.dot(pe_ref[...], wp_ref[...], preferred_element_type=jnp.float32))

    return pl.pallas_call(
        body,
        out_shape=jax.ShapeDtypeStruct((_N, _HID), jnp.float32),
    )(x, pe, wn, wp)


def _qkv(h, wq, wkv):
    def body(h_ref, wq_ref, wkv_ref, q_ref, kv_ref):
        hh = h_ref[...]
        q_ref[...] = jnp.dot(hh, wq_ref[...], preferred_element_type=jnp.float32)
        kv_ref[...] = jnp.dot(hh, wkv_ref[...], preferred_element_type=jnp.float32)

    return pl.pallas_call(
        body,
        out_shape=[
            jax.ShapeDtypeStruct((_N, _HID), jnp.float32),
            jax.ShapeDtypeStruct((_N, 2 * _HID), jnp.float32),
        ],
    )(h, wq, wkv)


def _final(h, batch2d, wm1, bm1, wm2, bm2, wv1, bv1, wv2, bv2):
    def body(h_ref, b_ref, wm1_ref, bm1_ref, wm2_ref, bm2_ref, wv1_ref, bv1_ref,
             wv2_ref, bv2_ref, mu_ref, std_ref):
        oh = (lax.broadcasted_iota(jnp.int32, (_G, _N), 0) == b_ref[...]).astype(
            jnp.float32)
        g = jnp.dot(oh, h_ref[...], preferred_element_type=jnp.float32)
        a1 = jnp.maximum(
            jnp.dot(g, wm1_ref[...], preferred_element_type=jnp.float32)
            + bm1_ref[...], 0.0)
        mu_ref[...] = (jnp.dot(a1, wm2_ref[...], preferred_element_type=jnp.float32)
                       + bm2_ref[...])
        a2 = jnp.maximum(
            jnp.dot(g, wv1_ref[...], preferred_element_type=jnp.float32)
            + bv1_ref[...], 0.0)
        lv = (jnp.dot(a2, wv2_ref[...], preferred_element_type=jnp.float32)
              + bv2_ref[...])
        std_ref[...] = jnp.exp(0.5 * lv)

    return pl.pallas_call(
        body,
        out_shape=[
            jax.ShapeDtypeStruct((_G, 1), jnp.float32),
            jax.ShapeDtypeStruct((_G, 1), jnp.float32),
        ],
    )(h, batch2d, wm1, bm1, wm2, bm2, wv1, bv1, wv2, bv2)


def kernel(x, edge_index, edge_attr, pe, batch, W_node, W_edge, W_pe, Wq, Wk,
           Wv, We, Wo, Woe, n1_g, n1_b, n2_g, n2_b, e1_g, e1_b, Wff1, bff1,
           Wff2, bff2, Wm1, bm1, Wm2, bm2, Wv1, bv1, Wv2, bv2):
    src = edge_index[0]
    dst = edge_index[1]
    row = lambda a: a.reshape(1, -1)

    h = _embed(x, pe, W_node, W_pe)
    estate = edge_attr
    escale = None
    L = Wq.shape[0]
    for l in range(L):
        first = l == 0
        last = l == L - 1
        q, kv = _qkv(h, Wq[l], jnp.concatenate([Wk[l], Wv[l]], axis=1))
        qd, kvs = _sc_gather(q, kv, dst, src)
        ew = W_edge if first else escale
        outs = _edge(first, last, estate, qd, kvs, ew, We[l],
                     None if last else Woe[l])
        msg, wpad = outs[0], outs[1]
        agg, den = _sc_scatter(msg, wpad, dst)
        stats = None if last else outs[3]
        h, escale = _node(h, agg, den, stats, Wo[l], row(n1_g[l]), row(n1_b[l]),
                          Wff1[l], row(bff1[l]), Wff2[l], row(bff2[l]),
                          row(n2_g[l]), row(n2_b[l]),
                          None if last else row(e1_g[l]),
                          None if last else row(e1_b[l]))
        if not last:
            estate = outs[2]

    return _final(h, batch.reshape(1, _N).astype(jnp.int32), Wm1, row(bm1),
                  Wm2, bm2.reshape(1, 1), Wv1, row(bv1), Wv2, bv2.reshape(1, 1))


# speedup vs baseline: 1.0608x; 1.0608x over previous
import functools

import jax
import jax.numpy as jnp
import numpy as np
from jax import lax
from jax.experimental import pallas as pl
from jax.experimental.pallas import tpu as pltpu
from jax.experimental.pallas import tpu_sc as plsc

L = 4
NH = 8
DH = 32
HID = 256
G = 16
_NC = 2
_NS = 16
_KCH = 128
_BN = 1000
_BEM = 800


def _sc_mesh():
    return plsc.VectorSubcoreMesh(core_axis_name="c", subcore_axis_name="s")


def _mm(h, w, bn):
    def body(h_ref, w_ref, o_ref):
        o_ref[...] = jnp.dot(h_ref[...], w_ref[...],
                             preferred_element_type=jnp.float32)

    return pl.pallas_call(
        body,
        grid=(h.shape[0] // bn,),
        in_specs=[
            pl.BlockSpec((bn, h.shape[1]), lambda b: (b, 0)),
            pl.BlockSpec(w.shape, lambda b: (0, 0)),
        ],
        out_specs=pl.BlockSpec((bn, w.shape[1]), lambda b: (b, 0)),
        out_shape=jax.ShapeDtypeStruct((h.shape[0], w.shape[1]), jnp.float32),
        compiler_params=pltpu.CompilerParams(
            dimension_semantics=("parallel",),
        ),
    )(h, w)


def _sc_gather(q, kv, dst, src):
    e = dst.shape[0]
    nchunk = e // _KCH

    @functools.partial(
        pl.kernel,
        out_type=(
            jax.ShapeDtypeStruct((e, HID), jnp.float32),
            jax.ShapeDtypeStruct((e, 2 * HID), jnp.float32),
        ),
        mesh=_sc_mesh(),
        scratch_types=[
            pltpu.VMEM((_KCH,), jnp.int32),
            pltpu.VMEM((_KCH,), jnp.int32),
            pltpu.VMEM((_KCH, HID), jnp.float32),
            pltpu.VMEM((_KCH, 2 * HID), jnp.float32),
            pltpu.SemaphoreType.DMA,
            pltpu.SemaphoreType.DMA,
        ],
    )
    def k(q_hbm, kv_hbm, dst_hbm, src_hbm, qd_hbm, kvs_hbm, di, si, bq, bkv,
          s1, s2):
        wid = lax.axis_index("s") * _NC + lax.axis_index("c")
        nper = (nchunk + _NC * _NS - 1) // (_NC * _NS)

        @pl.loop(0, nper)
        def _(j):
            g = j * (_NC * _NS) + wid

            @pl.when(g < nchunk)
            def _():
                base = g * _KCH
                pltpu.sync_copy(dst_hbm.at[pl.ds(base, _KCH)], di)
                pltpu.sync_copy(src_hbm.at[pl.ds(base, _KCH)], si)
                cq = pltpu.async_copy(q_hbm.at[di], bq, s1)
                ckv = pltpu.async_copy(kv_hbm.at[si], bkv, s2)
                cq.wait()
                ckv.wait()
                pltpu.sync_copy(bq, qd_hbm.at[pl.ds(base, _KCH)])
                pltpu.sync_copy(bkv, kvs_hbm.at[pl.ds(base, _KCH)])

    return k(q, kv, dst, src)


def _bn(x, g, b):
    m = x.mean(axis=0)
    v = x.var(axis=0)
    return (x - m) / jnp.sqrt(v + 1e-5) * g + b


def _seg_softmax(s, seg, n):
    m = jax.ops.segment_max(s, seg, num_segments=n)
    ex = jnp.exp(s - m[seg])
    den = jax.ops.segment_sum(ex, seg, num_segments=n)
    return ex / (den[seg] + 1e-16)


def _gt_layer(h, e, src, dst, Wq, Wk, Wv, We, Wo, Woe, n1g, n1b, n2g, n2b,
              e1g, e1b, Wf1, bf1, Wf2, bf2):
    n = h.shape[0]
    q = _mm(h, Wq, _BN)
    kv = jnp.concatenate([_mm(h, Wk, _BN), _mm(h, Wv, _BN)], axis=1)
    qd, kvs = _sc_gather(q, kv, dst, src)
    qd = qd.reshape(-1, NH, DH)
    ks = kvs[:, :HID].reshape(-1, NH, DH)
    vs = kvs[:, HID:].reshape(-1, NH, DH)
    ee = _mm(e, We, _BEM).reshape(-1, NH, DH)
    score = qd * ks / jnp.sqrt(jnp.float32(DH))
    score = score * ee
    e_new = score.reshape(-1, HID)
    s = jnp.clip(score.sum(-1), -5.0, 5.0)
    attn = _seg_softmax(s, dst, n)
    msg = attn[:, :, None] * vs
    agg = jax.ops.segment_sum(msg, dst, num_segments=n).reshape(n, HID)
    h = _bn(h + _mm(agg, Wo, _BN), n1g, n1b)
    ff = _mm(jax.nn.relu(_mm(h, Wf1, _BN) + bf1), Wf2, _BN) + bf2
    h = _bn(h + ff, n2g, n2b)
    e = _bn(e + _mm(e_new, Woe, _BEM), e1g, e1b)
    return h, e


def kernel(x, edge_index, edge_attr, pe, batch, W_node, W_edge, W_pe, Wq, Wk,
           Wv, We, Wo, Woe, n1_g, n1_b, n2_g, n2_b, e1_g, e1_b, Wff1, bff1,
           Wff2, bff2, Wm1, bm1, Wm2, bm2, Wv1, bv1, Wv2, bv2):
    src = edge_index[0]
    dst = edge_index[1]
    h = _mm(x, W_node, _BN) + pe @ W_pe
    e = edge_attr @ W_edge
    for l in range(L):
        h, e = _gt_layer(h, e, src, dst, Wq[l], Wk[l], Wv[l], We[l], Wo[l],
                         Woe[l], n1_g[l], n1_b[l], n2_g[l], n2_b[l], e1_g[l],
                         e1_b[l], Wff1[l], bff1[l], Wff2[l], bff2[l])
    g = jax.ops.segment_sum(h, batch, num_segments=G)
    mu = jax.nn.relu(g @ Wm1 + bm1) @ Wm2 + bm2
    log_var = jax.nn.relu(g @ Wv1 + bv1) @ Wv2 + bv2
    std = jnp.exp(0.5 * log_var)
    return (mu, std)
